# baseline (speedup 1.0000x reference)
import functools

import jax
import jax.numpy as jnp
from jax import lax
from jax.experimental import pallas as pl
from jax.experimental.pallas import tpu as pltpu
from jax.experimental.pallas import tpu_sc as plsc

B = 16384
GMF = 16
MLP = 32
NC = 2
NS = 16
NW = NC * NS
BPW = B // NW


@functools.cache
def _make_sc_gather():
    mesh = plsc.VectorSubcoreMesh(core_axis_name="c", subcore_axis_name="s")

    @functools.partial(
        pl.kernel,
        out_type=[
            jax.ShapeDtypeStruct((B, GMF), jnp.float32),
            jax.ShapeDtypeStruct((B, GMF), jnp.float32),
            jax.ShapeDtypeStruct((B, MLP), jnp.float32),
            jax.ShapeDtypeStruct((B, MLP), jnp.float32),
        ],
        mesh=mesh,
        scratch_types=[
            pltpu.VMEM((BPW,), jnp.int32),
            pltpu.VMEM((BPW,), jnp.int32),
            pltpu.VMEM((BPW, GMF), jnp.float32),
            pltpu.VMEM((BPW, GMF), jnp.float32),
            pltpu.VMEM((BPW, MLP), jnp.float32),
            pltpu.VMEM((BPW, MLP), jnp.float32),
            pltpu.SemaphoreType.DMA,
        ],
        compiler_params=pltpu.CompilerParams(use_tc_tiling_on_sc=False),
    )
    def _sc_gather(user_hbm, item_hbm, gu_t, gi_t, mu_t, mi_t,
                   out_gu, out_gi, out_mu, out_mi,
                   idx_u, idx_i, gu_v, gi_v, mu_v, mi_v, sem):
        wid = lax.axis_index("s") * NC + lax.axis_index("c")
        base = wid * BPW
        pltpu.sync_copy(user_hbm.at[pl.ds(base, BPW)], idx_u)
        pltpu.sync_copy(item_hbm.at[pl.ds(base, BPW)], idx_i)
        c1 = pltpu.async_copy(gu_t.at[idx_u], gu_v, sem)
        c2 = pltpu.async_copy(gi_t.at[idx_i], gi_v, sem)
        c3 = pltpu.async_copy(mu_t.at[idx_u], mu_v, sem)
        c4 = pltpu.async_copy(mi_t.at[idx_i], mi_v, sem)
        c1.wait()
        c2.wait()
        c3.wait()
        c4.wait()
        pltpu.sync_copy(gu_v, out_gu.at[pl.ds(base, BPW)])
        pltpu.sync_copy(gi_v, out_gi.at[pl.ds(base, BPW)])
        pltpu.sync_copy(mu_v, out_mu.at[pl.ds(base, BPW)])
        pltpu.sync_copy(mi_v, out_mi.at[pl.ds(base, BPW)])

    return _sc_gather


BLK = 2048


def _mlp_body(gu_ref, gi_ref, mu_ref, mi_ref,
              w1a_ref, w1b_ref, b1_ref, w2_ref, b2_ref, w3_ref, b3_ref,
              wog_ref, woh_ref, bo_ref, out_ref):
    h = jnp.dot(mu_ref[...], w1a_ref[...], preferred_element_type=jnp.float32)
    h = h + jnp.dot(mi_ref[...], w1b_ref[...], preferred_element_type=jnp.float32)
    h = jnp.maximum(h + b1_ref[...], 0.0)
    h = jnp.maximum(
        jnp.dot(h, w2_ref[...], preferred_element_type=jnp.float32) + b2_ref[...], 0.0)
    h = jnp.maximum(
        jnp.dot(h, w3_ref[...], preferred_element_type=jnp.float32) + b3_ref[...], 0.0)
    g = gu_ref[...] * gi_ref[...]
    out = (jnp.sum(g * wog_ref[...], axis=1, keepdims=True)
           + jnp.sum(h * woh_ref[...], axis=1, keepdims=True)
           + bo_ref[...])
    out_ref[...] = out


def _tc_mlp(gu, gi, mu, mi, W1a, W1b, b1r, W2, b2r, W3, b3r, wog, woh, bor):
    grid = (B // BLK,)
    row = lambda i: (i, 0)
    rep = lambda i: (0, 0)
    return pl.pallas_call(
        _mlp_body,
        grid=grid,
        in_specs=[
            pl.BlockSpec((BLK, GMF), row),
            pl.BlockSpec((BLK, GMF), row),
            pl.BlockSpec((BLK, MLP), row),
            pl.BlockSpec((BLK, MLP), row),
            pl.BlockSpec((MLP, 64), rep),
            pl.BlockSpec((MLP, 64), rep),
            pl.BlockSpec((1, 64), rep),
            pl.BlockSpec((64, 32), rep),
            pl.BlockSpec((1, 32), rep),
            pl.BlockSpec((32, 16), rep),
            pl.BlockSpec((1, 16), rep),
            pl.BlockSpec((1, GMF), rep),
            pl.BlockSpec((1, 16), rep),
            pl.BlockSpec((1, 1), rep),
        ],
        out_specs=pl.BlockSpec((BLK, 1), row),
        out_shape=jax.ShapeDtypeStruct((B, 1), jnp.float32),
    )(gu, gi, mu, mi, W1a, W1b, b1r, W2, b2r, W3, b3r, wog, woh, bor)


def kernel(user, item, gmf_user, gmf_item, mlp_user, mlp_item,
           W1, b1, W2, b2, W3, b3, Wo, bo):
    user = user.astype(jnp.int32)
    item = item.astype(jnp.int32)
    gu, gi, mu, mi = _make_sc_gather()(user, item, gmf_user, gmf_item,
                                       mlp_user, mlp_item)
    out = _tc_mlp(
        gu, gi, mu, mi,
        W1[:MLP], W1[MLP:], b1.reshape(1, -1),
        W2, b2.reshape(1, -1),
        W3, b3.reshape(1, -1),
        Wo[:GMF, 0].reshape(1, -1), Wo[GMF:, 0].reshape(1, -1),
        bo.reshape(1, 1),
    )
    return out[:, 0]
